# Initial kernel scaffold; baseline (speedup 1.0000x reference)
import jax
import jax.numpy as jnp
from jax.experimental import pallas as pl


def kernel(x, edge_index, W_ih, W_hh, b_ih, b_hh, W_self, b_self, W_neigh, b_neigh):
    raise NotImplementedError("write your pallas kernel here")


# speedup vs baseline: 3.4668x; 3.4668x over previous
import functools

import jax
import jax.numpy as jnp
from jax import lax
from jax.experimental import pallas as pl
from jax.experimental.pallas import tpu as pltpu
from jax.experimental.pallas import tpu_sc as plsc

N_NODES = 10000
K_NEI = 16
D_FEAT = 128
L_LAYERS = 4

_NC = 2
_NS = 16
_NW = _NC * _NS

_R_EDGES = N_NODES * K_NEI
_PER_W = _R_EDGES // _NW
_CHUNK = 50
_N_CHUNKS = _PER_W // _CHUNK
_GROUP = 5
_N_GROUPS = _N_CHUNKS // _GROUP
_TOT_CHUNKS = _R_EDGES // _CHUNK


def _sc_gather(table, idx3):
    mesh = plsc.VectorSubcoreMesh(core_axis_name="c", subcore_axis_name="s")

    @functools.partial(
        pl.kernel,
        mesh=mesh,
        out_type=jax.ShapeDtypeStruct((_TOT_CHUNKS, _CHUNK, D_FEAT), jnp.float32),
        scratch_types=[
            pltpu.VMEM((_N_CHUNKS, _CHUNK), jnp.int32),
            pltpu.VMEM((_GROUP, _CHUNK, D_FEAT), jnp.float32),
            pltpu.SemaphoreType.DMA,
        ],
    )
    def gather_kernel(table_hbm, idx_hbm, out_hbm, idx_v, rows_v, sem):
        wid = lax.axis_index("s") * _NC + lax.axis_index("c")
        cbase = wid * _N_CHUNKS
        pltpu.sync_copy(idx_hbm.at[wid], idx_v)

        def group_body(g, carry):
            handles = [
                pltpu.async_copy(
                    table_hbm.at[idx_v.at[g * _GROUP + j]], rows_v.at[j], sem)
                for j in range(_GROUP)
            ]
            for h in handles:
                h.wait()
            pltpu.sync_copy(rows_v, out_hbm.at[pl.ds(cbase + g * _GROUP, _GROUP)])
            return carry

        lax.fori_loop(0, _N_GROUPS, group_body, 0)

    return gather_kernel(table, idx3)


def _make_lstm_body(bn, relu):
    def body(m_ref, h_ref, wih_ref, whh_ref, b_ref, ws_ref, wn_ref, bo_ref,
             out_ref, xg_ref):
        xflat = m_ref[...].reshape(K_NEI * bn, D_FEAT)
        xg_ref[...] = (
            jnp.dot(xflat, wih_ref[...], preferred_element_type=jnp.float32)
            + b_ref[...]
        )

        g0 = xg_ref[pl.ds(0, bn), :]
        i = jax.nn.sigmoid(g0[:, :D_FEAT])
        gg = jnp.tanh(g0[:, 2 * D_FEAT:3 * D_FEAT])
        o = jax.nn.sigmoid(g0[:, 3 * D_FEAT:])
        cp = i * gg
        hp = o * jnp.tanh(cp)

        for t in range(1, K_NEI):
            g = xg_ref[pl.ds(t * bn, bn), :] + jnp.dot(
                hp, whh_ref[...], preferred_element_type=jnp.float32)
            i = jax.nn.sigmoid(g[:, :D_FEAT])
            f = jax.nn.sigmoid(g[:, D_FEAT:2 * D_FEAT])
            gg = jnp.tanh(g[:, 2 * D_FEAT:3 * D_FEAT])
            o = jax.nn.sigmoid(g[:, 3 * D_FEAT:])
            cp = f * cp + i * gg
            hp = o * jnp.tanh(cp)

        out = (
            jnp.dot(h_ref[...], ws_ref[...], preferred_element_type=jnp.float32)
            + jnp.dot(hp, wn_ref[...], preferred_element_type=jnp.float32)
            + bo_ref[...]
        )
        if relu:
            out = jnp.maximum(out, 0.0)
        out_ref[...] = out

    return body


def _tc_layer(m_knd, h, wihT, whhT, b2, wsT, wnT, bo2, relu, bn=400):
    grid = (N_NODES // bn,)
    full = lambda j: (0, 0)
    return pl.pallas_call(
        _make_lstm_body(bn, relu),
        grid=grid,
        in_specs=[
            pl.BlockSpec((K_NEI, bn, D_FEAT), lambda j: (0, j, 0)),
            pl.BlockSpec((bn, D_FEAT), lambda j: (j, 0)),
            pl.BlockSpec((D_FEAT, 4 * D_FEAT), full),
            pl.BlockSpec((D_FEAT, 4 * D_FEAT), full),
            pl.BlockSpec((1, 4 * D_FEAT), full),
            pl.BlockSpec((D_FEAT, D_FEAT), full),
            pl.BlockSpec((D_FEAT, D_FEAT), full),
            pl.BlockSpec((1, D_FEAT), full),
        ],
        out_specs=pl.BlockSpec((bn, D_FEAT), lambda j: (j, 0)),
        out_shape=jax.ShapeDtypeStruct((N_NODES, D_FEAT), jnp.float32),
        scratch_shapes=[pltpu.VMEM((K_NEI * bn, 4 * D_FEAT), jnp.float32)],
        compiler_params=pltpu.CompilerParams(
            dimension_semantics=("arbitrary",)),
    )(m_knd, h, wihT, whhT, b2, wsT, wnT, bo2)


def kernel(x, edge_index, W_ih, W_hh, b_ih, b_hh, W_self, b_self, W_neigh, b_neigh):
    src = edge_index[0]
    idx3 = src.reshape(N_NODES, K_NEI).T.reshape(_NW, _N_CHUNKS, _CHUNK)

    wihT = jnp.transpose(W_ih, (0, 2, 1))
    whhT = jnp.transpose(W_hh, (0, 2, 1))
    b2 = (b_ih + b_hh).reshape(L_LAYERS, 1, 4 * D_FEAT)
    wsT = jnp.transpose(W_self, (0, 2, 1))
    wnT = jnp.transpose(W_neigh, (0, 2, 1))
    bo2 = (b_self + b_neigh).reshape(L_LAYERS, 1, D_FEAT)

    h = x
    for l in range(L_LAYERS):
        m = _sc_gather(h, idx3)
        m_knd = m.reshape(K_NEI, N_NODES, D_FEAT)
        h = _tc_layer(m_knd, h, wihT[l], whhT[l], b2[l], wsT[l], wnT[l],
                      bo2[l], relu=(l < L_LAYERS - 1))
    return h


# speedup vs baseline: 3.6037x; 1.0395x over previous
import functools

import jax
import jax.numpy as jnp
from jax import lax
from jax.experimental import pallas as pl
from jax.experimental.pallas import tpu as pltpu
from jax.experimental.pallas import tpu_sc as plsc

N_NODES = 10000
K_NEI = 16
D_FEAT = 128
L_LAYERS = 4

_NC = 2
_NS = 16
_NW = _NC * _NS

_R_EDGES = N_NODES * K_NEI
_PER_W = _R_EDGES // _NW
_CHUNK = 50
_N_CHUNKS = _PER_W // _CHUNK
_GROUP = 5
_N_GROUPS = _N_CHUNKS // _GROUP
_TOT_CHUNKS = _R_EDGES // _CHUNK


def _sc_gather(table, idx3):
    mesh = plsc.VectorSubcoreMesh(core_axis_name="c", subcore_axis_name="s")

    @functools.partial(
        pl.kernel,
        mesh=mesh,
        out_type=jax.ShapeDtypeStruct((_TOT_CHUNKS, _CHUNK, D_FEAT), jnp.float32),
        scratch_types=[
            pltpu.VMEM((_N_CHUNKS, _CHUNK), jnp.int32),
            pltpu.VMEM((_GROUP, _CHUNK, D_FEAT), jnp.float32),
            pltpu.SemaphoreType.DMA,
        ],
    )
    def gather_kernel(table_hbm, idx_hbm, out_hbm, idx_v, rows_v, sem):
        wid = lax.axis_index("s") * _NC + lax.axis_index("c")
        cbase = wid * _N_CHUNKS
        pltpu.sync_copy(idx_hbm.at[wid], idx_v)

        def group_body(g, carry):
            handles = [
                pltpu.async_copy(
                    table_hbm.at[idx_v.at[g * _GROUP + j]], rows_v.at[j], sem)
                for j in range(_GROUP)
            ]
            for h in handles:
                h.wait()
            pltpu.sync_copy(rows_v, out_hbm.at[pl.ds(cbase + g * _GROUP, _GROUP)])
            return carry

        lax.fori_loop(0, _N_GROUPS, group_body, 0)

    return gather_kernel(table, idx3)


def _sigmoid(x):
    return 0.5 * jnp.tanh(0.5 * x) + 0.5


def _make_lstm_body(bn, relu):
    def body(m_ref, h_ref, wih_ref, whh_ref, b_ref, ws_ref, wn_ref, bo_ref,
             out_ref, xg_ref):
        xflat = m_ref[...].reshape(K_NEI * bn, D_FEAT)
        xg_ref[...] = (
            jnp.dot(xflat, wih_ref[...], preferred_element_type=jnp.float32)
            + b_ref[...]
        )

        g0 = xg_ref[pl.ds(0, bn), :]
        i = _sigmoid(g0[:, :D_FEAT])
        gg = jnp.tanh(g0[:, 2 * D_FEAT:3 * D_FEAT])
        o = _sigmoid(g0[:, 3 * D_FEAT:])
        cp = i * gg
        hp = o * jnp.tanh(cp)

        for t in range(1, K_NEI):
            g = xg_ref[pl.ds(t * bn, bn), :] + jnp.dot(
                hp, whh_ref[...], preferred_element_type=jnp.float32)
            i = _sigmoid(g[:, :D_FEAT])
            f = _sigmoid(g[:, D_FEAT:2 * D_FEAT])
            gg = jnp.tanh(g[:, 2 * D_FEAT:3 * D_FEAT])
            o = _sigmoid(g[:, 3 * D_FEAT:])
            cp = f * cp + i * gg
            hp = o * jnp.tanh(cp)

        out = (
            jnp.dot(h_ref[...], ws_ref[...], preferred_element_type=jnp.float32)
            + jnp.dot(hp, wn_ref[...], preferred_element_type=jnp.float32)
            + bo_ref[...]
        )
        if relu:
            out = jnp.maximum(out, 0.0)
        out_ref[...] = out

    return body


def _tc_layer(m_knd, h, wihT, whhT, b2, wsT, wnT, bo2, relu, bn=400):
    grid = (N_NODES // bn,)
    full = lambda j: (0, 0)
    return pl.pallas_call(
        _make_lstm_body(bn, relu),
        grid=grid,
        in_specs=[
            pl.BlockSpec((K_NEI, bn, D_FEAT), lambda j: (0, j, 0)),
            pl.BlockSpec((bn, D_FEAT), lambda j: (j, 0)),
            pl.BlockSpec((D_FEAT, 4 * D_FEAT), full),
            pl.BlockSpec((D_FEAT, 4 * D_FEAT), full),
            pl.BlockSpec((1, 4 * D_FEAT), full),
            pl.BlockSpec((D_FEAT, D_FEAT), full),
            pl.BlockSpec((D_FEAT, D_FEAT), full),
            pl.BlockSpec((1, D_FEAT), full),
        ],
        out_specs=pl.BlockSpec((bn, D_FEAT), lambda j: (j, 0)),
        out_shape=jax.ShapeDtypeStruct((N_NODES, D_FEAT), jnp.float32),
        scratch_shapes=[pltpu.VMEM((K_NEI * bn, 4 * D_FEAT), jnp.float32)],
        compiler_params=pltpu.CompilerParams(
            dimension_semantics=("arbitrary",)),
    )(m_knd, h, wihT, whhT, b2, wsT, wnT, bo2)


def kernel(x, edge_index, W_ih, W_hh, b_ih, b_hh, W_self, b_self, W_neigh, b_neigh):
    src = edge_index[0]
    idx3 = src.reshape(N_NODES, K_NEI).T.reshape(_NW, _N_CHUNKS, _CHUNK)

    wihT = jnp.transpose(W_ih, (0, 2, 1))
    whhT = jnp.transpose(W_hh, (0, 2, 1))
    b2 = (b_ih + b_hh).reshape(L_LAYERS, 1, 4 * D_FEAT)
    wsT = jnp.transpose(W_self, (0, 2, 1))
    wnT = jnp.transpose(W_neigh, (0, 2, 1))
    bo2 = (b_self + b_neigh).reshape(L_LAYERS, 1, D_FEAT)

    h = x
    for l in range(L_LAYERS):
        m = _sc_gather(h, idx3)
        m_knd = m.reshape(K_NEI, N_NODES, D_FEAT)
        h = _tc_layer(m_knd, h, wihT[l], whhT[l], b2[l], wsT[l], wnT[l],
                      bo2[l], relu=(l < L_LAYERS - 1))
    return h


# speedup vs baseline: 3.6176x; 1.0039x over previous
import functools

import jax
import jax.numpy as jnp
from jax import lax
from jax.experimental import pallas as pl
from jax.experimental.pallas import tpu as pltpu
from jax.experimental.pallas import tpu_sc as plsc

N_NODES = 10000
K_NEI = 16
D_FEAT = 128
L_LAYERS = 4

_NC = 2
_NS = 16
_NW = _NC * _NS

_R_EDGES = N_NODES * K_NEI
_PER_W = _R_EDGES // _NW
_CHUNK = 50
_N_CHUNKS = _PER_W // _CHUNK
_GROUP = 5
_N_GROUPS = _N_CHUNKS // _GROUP
_TOT_CHUNKS = _R_EDGES // _CHUNK


def _sc_gather(table, idx3):
    mesh = plsc.VectorSubcoreMesh(core_axis_name="c", subcore_axis_name="s")

    @functools.partial(
        pl.kernel,
        mesh=mesh,
        out_type=jax.ShapeDtypeStruct((_TOT_CHUNKS, _CHUNK, D_FEAT), jnp.float32),
        scratch_types=[
            pltpu.VMEM((_N_CHUNKS, _CHUNK), jnp.int32),
            pltpu.VMEM((_GROUP, _CHUNK, D_FEAT), jnp.float32),
            pltpu.SemaphoreType.DMA,
        ],
    )
    def gather_kernel(table_hbm, idx_hbm, out_hbm, idx_v, rows_v, sem):
        wid = lax.axis_index("s") * _NC + lax.axis_index("c")
        cbase = wid * _N_CHUNKS
        pltpu.sync_copy(idx_hbm.at[wid], idx_v)

        def group_body(g, carry):
            handles = [
                pltpu.async_copy(
                    table_hbm.at[idx_v.at[g * _GROUP + j]], rows_v.at[j], sem)
                for j in range(_GROUP)
            ]
            for h in handles:
                h.wait()
            pltpu.sync_copy(rows_v, out_hbm.at[pl.ds(cbase + g * _GROUP, _GROUP)])
            return carry

        lax.fori_loop(0, _N_GROUPS, group_body, 0)

    return gather_kernel(table, idx3)


def _sigmoid(x):
    return 0.5 * jnp.tanh(0.5 * x) + 0.5


def _make_lstm_body(bn, relu):
    def body(m_ref, h_ref, wih_ref, whh_ref, b_ref, ws_ref, wn_ref, bo_ref,
             out_ref, xg_ref):
        xflat = m_ref[...].reshape(K_NEI * bn, D_FEAT).astype(jnp.bfloat16)
        xg_ref[...] = (
            jnp.dot(xflat, wih_ref[...], preferred_element_type=jnp.float32)
            + b_ref[...]
        )

        g0 = xg_ref[pl.ds(0, bn), :]
        i = _sigmoid(g0[:, :D_FEAT])
        gg = jnp.tanh(g0[:, 2 * D_FEAT:3 * D_FEAT])
        o = _sigmoid(g0[:, 3 * D_FEAT:])
        cp = i * gg
        hp = o * jnp.tanh(cp)

        for t in range(1, K_NEI):
            g = xg_ref[pl.ds(t * bn, bn), :] + jnp.dot(
                hp.astype(jnp.bfloat16), whh_ref[...],
                preferred_element_type=jnp.float32)
            i = _sigmoid(g[:, :D_FEAT])
            f = _sigmoid(g[:, D_FEAT:2 * D_FEAT])
            gg = jnp.tanh(g[:, 2 * D_FEAT:3 * D_FEAT])
            o = _sigmoid(g[:, 3 * D_FEAT:])
            cp = f * cp + i * gg
            hp = o * jnp.tanh(cp)

        out = (
            jnp.dot(h_ref[...].astype(jnp.bfloat16), ws_ref[...],
                    preferred_element_type=jnp.float32)
            + jnp.dot(hp.astype(jnp.bfloat16), wn_ref[...],
                      preferred_element_type=jnp.float32)
            + bo_ref[...]
        )
        if relu:
            out = jnp.maximum(out, 0.0)
        out_ref[...] = out

    return body


def _tc_layer(m_knd, h, wihT, whhT, b2, wsT, wnT, bo2, relu, bn=400):
    grid = (N_NODES // bn,)
    full = lambda j: (0, 0)
    return pl.pallas_call(
        _make_lstm_body(bn, relu),
        grid=grid,
        in_specs=[
            pl.BlockSpec((K_NEI, bn, D_FEAT), lambda j: (0, j, 0)),
            pl.BlockSpec((bn, D_FEAT), lambda j: (j, 0)),
            pl.BlockSpec((D_FEAT, 4 * D_FEAT), full),
            pl.BlockSpec((D_FEAT, 4 * D_FEAT), full),
            pl.BlockSpec((1, 4 * D_FEAT), full),
            pl.BlockSpec((D_FEAT, D_FEAT), full),
            pl.BlockSpec((D_FEAT, D_FEAT), full),
            pl.BlockSpec((1, D_FEAT), full),
        ],
        out_specs=pl.BlockSpec((bn, D_FEAT), lambda j: (j, 0)),
        out_shape=jax.ShapeDtypeStruct((N_NODES, D_FEAT), jnp.float32),
        scratch_shapes=[pltpu.VMEM((K_NEI * bn, 4 * D_FEAT), jnp.float32)],
        compiler_params=pltpu.CompilerParams(
            dimension_semantics=("arbitrary",)),
    )(m_knd, h, wihT, whhT, b2, wsT, wnT, bo2)


def kernel(x, edge_index, W_ih, W_hh, b_ih, b_hh, W_self, b_self, W_neigh, b_neigh):
    src = edge_index[0]
    idx3 = src.reshape(N_NODES, K_NEI).T.reshape(_NW, _N_CHUNKS, _CHUNK)

    wihT = jnp.transpose(W_ih, (0, 2, 1)).astype(jnp.bfloat16)
    whhT = jnp.transpose(W_hh, (0, 2, 1)).astype(jnp.bfloat16)
    b2 = (b_ih + b_hh).reshape(L_LAYERS, 1, 4 * D_FEAT)
    wsT = jnp.transpose(W_self, (0, 2, 1)).astype(jnp.bfloat16)
    wnT = jnp.transpose(W_neigh, (0, 2, 1)).astype(jnp.bfloat16)
    bo2 = (b_self + b_neigh).reshape(L_LAYERS, 1, D_FEAT)

    h = x
    for l in range(L_LAYERS):
        m = _sc_gather(h, idx3)
        m_knd = m.reshape(K_NEI, N_NODES, D_FEAT)
        h = _tc_layer(m_knd, h, wihT[l], whhT[l], b2[l], wsT[l], wnT[l],
                      bo2[l], relu=(l < L_LAYERS - 1))
    return h


# speedup vs baseline: 4.9623x; 1.3717x over previous
import functools

import jax
import jax.numpy as jnp
from jax import lax
from jax.experimental import pallas as pl
from jax.experimental.pallas import tpu as pltpu
from jax.experimental.pallas import tpu_sc as plsc

N_NODES = 10000
K_NEI = 16
D_FEAT = 128
L_LAYERS = 4

_NC = 2
_NS = 16
_NW = _NC * _NS

_R_EDGES = N_NODES * K_NEI
_PER_W = _R_EDGES // _NW
_CHUNK = 40
_N_CHUNKS = _PER_W // _CHUNK
_GROUP = 5
_N_GROUPS = _N_CHUNKS // _GROUP
_TOT_CHUNKS = _R_EDGES // _CHUNK


def _sc_gather(table, idx3):
    mesh = plsc.VectorSubcoreMesh(core_axis_name="c", subcore_axis_name="s")

    @functools.partial(
        pl.kernel,
        mesh=mesh,
        out_type=jax.ShapeDtypeStruct((_TOT_CHUNKS, _CHUNK, D_FEAT), jnp.float32),
        scratch_types=[
            pltpu.VMEM((_N_CHUNKS, _CHUNK), jnp.int32),
            pltpu.VMEM((_GROUP, _CHUNK, D_FEAT), jnp.float32),
            pltpu.SemaphoreType.DMA,
        ],
    )
    def gather_kernel(table_hbm, idx_hbm, out_hbm, idx_v, rows_v, sem):
        wid = lax.axis_index("s") * _NC + lax.axis_index("c")
        cbase = wid * _N_CHUNKS
        pltpu.sync_copy(idx_hbm.at[wid], idx_v)

        def group_body(g, carry):
            handles = [
                pltpu.async_copy(
                    table_hbm.at[idx_v.at[g * _GROUP + j]], rows_v.at[j], sem)
                for j in range(_GROUP)
            ]
            for h in handles:
                h.wait()
            pltpu.sync_copy(rows_v, out_hbm.at[pl.ds(cbase + g * _GROUP, _GROUP)])
            return carry

        lax.fori_loop(0, _N_GROUPS, group_body, 0)

    return gather_kernel(table, idx3)


def _sigmoid(x):
    return 0.5 * jnp.tanh(0.5 * x) + 0.5


def _make_lstm_body(bn, relu):
    def body(m_ref, h_ref, wih_ref, whh_ref, b_ref, ws_ref, wn_ref, bo_ref,
             out_ref, xg_ref):
        xflat = m_ref[...].reshape(K_NEI * bn, D_FEAT).astype(jnp.bfloat16)
        xg_ref[...] = (
            jnp.dot(xflat, wih_ref[...], preferred_element_type=jnp.float32)
            + b_ref[...]
        )

        g0 = xg_ref[pl.ds(0, bn), :]
        i = _sigmoid(g0[:, :D_FEAT])
        gg = jnp.tanh(g0[:, 2 * D_FEAT:3 * D_FEAT])
        o = _sigmoid(g0[:, 3 * D_FEAT:])
        cp = i * gg
        hp = o * jnp.tanh(cp)

        for t in range(1, K_NEI):
            g = xg_ref[pl.ds(t * bn, bn), :] + jnp.dot(
                hp.astype(jnp.bfloat16), whh_ref[...],
                preferred_element_type=jnp.float32)
            i = _sigmoid(g[:, :D_FEAT])
            f = _sigmoid(g[:, D_FEAT:2 * D_FEAT])
            gg = jnp.tanh(g[:, 2 * D_FEAT:3 * D_FEAT])
            o = _sigmoid(g[:, 3 * D_FEAT:])
            cp = f * cp + i * gg
            hp = o * jnp.tanh(cp)

        out = (
            jnp.dot(h_ref[...].astype(jnp.bfloat16), ws_ref[...],
                    preferred_element_type=jnp.float32)
            + jnp.dot(hp.astype(jnp.bfloat16), wn_ref[...],
                      preferred_element_type=jnp.float32)
            + bo_ref[...]
        )
        if relu:
            out = jnp.maximum(out, 0.0)
        out_ref[...] = out

    return body


def _tc_layer(m_knd, h, wihT, whhT, b2, wsT, wnT, bo2, relu, bn=400):
    grid = (N_NODES // bn,)
    full = lambda j: (0, 0)
    return pl.pallas_call(
        _make_lstm_body(bn, relu),
        grid=grid,
        in_specs=[
            pl.BlockSpec((K_NEI, bn, D_FEAT), lambda j: (0, j, 0)),
            pl.BlockSpec((bn, D_FEAT), lambda j: (j, 0)),
            pl.BlockSpec((D_FEAT, 4 * D_FEAT), full),
            pl.BlockSpec((D_FEAT, 4 * D_FEAT), full),
            pl.BlockSpec((1, 4 * D_FEAT), full),
            pl.BlockSpec((D_FEAT, D_FEAT), full),
            pl.BlockSpec((D_FEAT, D_FEAT), full),
            pl.BlockSpec((1, D_FEAT), full),
        ],
        out_specs=pl.BlockSpec((bn, D_FEAT), lambda j: (j, 0)),
        out_shape=jax.ShapeDtypeStruct((N_NODES, D_FEAT), jnp.float32),
        scratch_shapes=[pltpu.VMEM((K_NEI * bn, 4 * D_FEAT), jnp.float32)],
        compiler_params=pltpu.CompilerParams(
            dimension_semantics=("arbitrary",)),
    )(m_knd, h, wihT, whhT, b2, wsT, wnT, bo2)


def kernel(x, edge_index, W_ih, W_hh, b_ih, b_hh, W_self, b_self, W_neigh, b_neigh):
    src = edge_index[0]
    idx3 = src.reshape(N_NODES, K_NEI).T.reshape(_NW, _N_CHUNKS, _CHUNK)

    wihT = jnp.transpose(W_ih, (0, 2, 1)).astype(jnp.bfloat16)
    whhT = jnp.transpose(W_hh, (0, 2, 1)).astype(jnp.bfloat16)
    b2 = (b_ih + b_hh).reshape(L_LAYERS, 1, 4 * D_FEAT)
    wsT = jnp.transpose(W_self, (0, 2, 1)).astype(jnp.bfloat16)
    wnT = jnp.transpose(W_neigh, (0, 2, 1)).astype(jnp.bfloat16)
    bo2 = (b_self + b_neigh).reshape(L_LAYERS, 1, D_FEAT)

    h = x
    for l in range(L_LAYERS):
        m = _sc_gather(h, idx3)
        m_knd = m.reshape(K_NEI, N_NODES, D_FEAT)
        h = _tc_layer(m_knd, h, wihT[l], whhT[l], b2[l], wsT[l], wnT[l],
                      bo2[l], relu=(l < L_LAYERS - 1))
    return h


# speedup vs baseline: 5.9782x; 1.2047x over previous
import functools

import jax
import jax.numpy as jnp
from jax import lax
from jax.experimental import pallas as pl
from jax.experimental.pallas import tpu as pltpu
from jax.experimental.pallas import tpu_sc as plsc

N_NODES = 10000
K_NEI = 16
D_FEAT = 128
L_LAYERS = 4

_NC = 2
_NS = 16
_NW = _NC * _NS

_NCH = 5
_NODES_C = N_NODES // _NCH
_R_EDGES = _NODES_C * K_NEI
_PER_W = _R_EDGES // _NW
_CHUNK = 40
_N_CHUNKS = _PER_W // _CHUNK
_GROUP = 5
_N_GROUPS = _N_CHUNKS // _GROUP
_TOT_CHUNKS = _R_EDGES // _CHUNK


def _sc_gather(table, idx3):
    mesh = plsc.VectorSubcoreMesh(core_axis_name="c", subcore_axis_name="s")

    @functools.partial(
        pl.kernel,
        mesh=mesh,
        out_type=jax.ShapeDtypeStruct((_TOT_CHUNKS, _CHUNK, D_FEAT), jnp.float32),
        scratch_types=[
            pltpu.VMEM((_N_CHUNKS, _CHUNK), jnp.int32),
            pltpu.VMEM((_GROUP, _CHUNK, D_FEAT), jnp.float32),
            pltpu.SemaphoreType.DMA,
        ],
    )
    def gather_kernel(table_hbm, idx_hbm, out_hbm, idx_v, rows_v, sem):
        wid = lax.axis_index("s") * _NC + lax.axis_index("c")
        cbase = wid * _N_CHUNKS
        pltpu.sync_copy(idx_hbm.at[wid], idx_v)

        def group_body(g, carry):
            handles = [
                pltpu.async_copy(
                    table_hbm.at[idx_v.at[g * _GROUP + j]], rows_v.at[j], sem)
                for j in range(_GROUP)
            ]
            for h in handles:
                h.wait()
            pltpu.sync_copy(rows_v, out_hbm.at[pl.ds(cbase + g * _GROUP, _GROUP)])
            return carry

        lax.fori_loop(0, _N_GROUPS, group_body, 0)

    return gather_kernel(table, idx3)


def _sigmoid(x):
    return 0.5 * jnp.tanh(0.5 * x) + 0.5


def _make_lstm_body(bn, relu):
    def body(m_ref, h_ref, wih_ref, whh_ref, b_ref, ws_ref, wn_ref, bo_ref,
             out_ref, xg_ref):
        xflat = m_ref[...].reshape(K_NEI * bn, D_FEAT).astype(jnp.bfloat16)
        xg_ref[...] = (
            jnp.dot(xflat, wih_ref[...], preferred_element_type=jnp.float32)
            + b_ref[...]
        )

        g0 = xg_ref[pl.ds(0, bn), :]
        i = _sigmoid(g0[:, :D_FEAT])
        gg = jnp.tanh(g0[:, 2 * D_FEAT:3 * D_FEAT])
        o = _sigmoid(g0[:, 3 * D_FEAT:])
        cp = i * gg
        hp = o * jnp.tanh(cp)

        for t in range(1, K_NEI):
            g = xg_ref[pl.ds(t * bn, bn), :] + jnp.dot(
                hp.astype(jnp.bfloat16), whh_ref[...],
                preferred_element_type=jnp.float32)
            i = _sigmoid(g[:, :D_FEAT])
            f = _sigmoid(g[:, D_FEAT:2 * D_FEAT])
            gg = jnp.tanh(g[:, 2 * D_FEAT:3 * D_FEAT])
            o = _sigmoid(g[:, 3 * D_FEAT:])
            cp = f * cp + i * gg
            hp = o * jnp.tanh(cp)

        out = (
            jnp.dot(h_ref[...].astype(jnp.bfloat16), ws_ref[...],
                    preferred_element_type=jnp.float32)
            + jnp.dot(hp.astype(jnp.bfloat16), wn_ref[...],
                      preferred_element_type=jnp.float32)
            + bo_ref[...]
        )
        if relu:
            out = jnp.maximum(out, 0.0)
        out_ref[...] = out

    return body


def _tc_layer(m_knd, h, wihT, whhT, b2, wsT, wnT, bo2, relu, bn=400):
    grid = (_NODES_C // bn,)
    full = lambda j: (0, 0)
    return pl.pallas_call(
        _make_lstm_body(bn, relu),
        grid=grid,
        in_specs=[
            pl.BlockSpec((K_NEI, bn, D_FEAT), lambda j: (0, j, 0)),
            pl.BlockSpec((bn, D_FEAT), lambda j: (j, 0)),
            pl.BlockSpec((D_FEAT, 4 * D_FEAT), full),
            pl.BlockSpec((D_FEAT, 4 * D_FEAT), full),
            pl.BlockSpec((1, 4 * D_FEAT), full),
            pl.BlockSpec((D_FEAT, D_FEAT), full),
            pl.BlockSpec((D_FEAT, D_FEAT), full),
            pl.BlockSpec((1, D_FEAT), full),
        ],
        out_specs=pl.BlockSpec((bn, D_FEAT), lambda j: (j, 0)),
        out_shape=jax.ShapeDtypeStruct((_NODES_C, D_FEAT), jnp.float32),
        scratch_shapes=[pltpu.VMEM((K_NEI * bn, 4 * D_FEAT), jnp.float32)],
        compiler_params=pltpu.CompilerParams(
            dimension_semantics=("arbitrary",)),
    )(m_knd, h, wihT, whhT, b2, wsT, wnT, bo2)


def kernel(x, edge_index, W_ih, W_hh, b_ih, b_hh, W_self, b_self, W_neigh, b_neigh):
    src = edge_index[0]
    idx4 = jnp.transpose(
        src.reshape(_NCH, _NODES_C, K_NEI), (0, 2, 1)
    ).reshape(_NCH, _NW, _N_CHUNKS, _CHUNK)

    wihT = jnp.transpose(W_ih, (0, 2, 1)).astype(jnp.bfloat16)
    whhT = jnp.transpose(W_hh, (0, 2, 1)).astype(jnp.bfloat16)
    b2 = (b_ih + b_hh).reshape(L_LAYERS, 1, 4 * D_FEAT)
    wsT = jnp.transpose(W_self, (0, 2, 1)).astype(jnp.bfloat16)
    wnT = jnp.transpose(W_neigh, (0, 2, 1)).astype(jnp.bfloat16)
    bo2 = (b_self + b_neigh).reshape(L_LAYERS, 1, D_FEAT)

    h = x
    for l in range(L_LAYERS):
        relu = l < L_LAYERS - 1
        ms = [_sc_gather(h, idx4[c]) for c in range(_NCH)]
        outs = []
        for c in range(_NCH):
            m_knd = ms[c].reshape(K_NEI, _NODES_C, D_FEAT)
            h_c = lax.slice(h, (c * _NODES_C, 0), ((c + 1) * _NODES_C, D_FEAT))
            outs.append(_tc_layer(m_knd, h_c, wihT[l], whhT[l], b2[l],
                                  wsT[l], wnT[l], bo2[l], relu=relu))
        h = jnp.concatenate(outs, axis=0)
    return h


# speedup vs baseline: 7.0779x; 1.1839x over previous
import functools

import jax
import jax.numpy as jnp
from jax import lax
from jax.experimental import pallas as pl
from jax.experimental.pallas import tpu as pltpu
from jax.experimental.pallas import tpu_sc as plsc

N_NODES = 10000
K_NEI = 16
D_FEAT = 128
L_LAYERS = 4

_NC = 2
_NS = 16
_NW = _NC * _NS

_NCH = 5
_NODES_C = N_NODES // _NCH
_R_EDGES = _NODES_C * K_NEI
_PER_W = _R_EDGES // _NW
_CHUNK = 40
_N_CHUNKS = _PER_W // _CHUNK
_GROUP = 5
_N_GROUPS = _N_CHUNKS // _GROUP
_TOT_CHUNKS = _R_EDGES // _CHUNK


def _sc_gather(table, idx3):
    mesh = plsc.VectorSubcoreMesh(core_axis_name="c", subcore_axis_name="s")

    @functools.partial(
        pl.kernel,
        mesh=mesh,
        out_type=jax.ShapeDtypeStruct((_TOT_CHUNKS, _CHUNK, D_FEAT), jnp.float32),
        scratch_types=[
            pltpu.VMEM((_N_CHUNKS, _CHUNK), jnp.int32),
            pltpu.VMEM((_GROUP, _CHUNK, D_FEAT), jnp.float32),
            pltpu.SemaphoreType.DMA,
        ],
    )
    def gather_kernel(table_hbm, idx_hbm, out_hbm, idx_v, rows_v, sem):
        wid = lax.axis_index("s") * _NC + lax.axis_index("c")
        cbase = wid * _N_CHUNKS
        pltpu.sync_copy(idx_hbm.at[wid], idx_v)

        def group_body(g, carry):
            handles = [
                pltpu.async_copy(
                    table_hbm.at[idx_v.at[g * _GROUP + j]], rows_v.at[j], sem)
                for j in range(_GROUP)
            ]
            for h in handles:
                h.wait()
            pltpu.sync_copy(rows_v, out_hbm.at[pl.ds(cbase + g * _GROUP, _GROUP)])
            return carry

        lax.fori_loop(0, _N_GROUPS, group_body, 0)

    return gather_kernel(table, idx3)


def _make_lstm_body(bn, relu):
    def body(m_ref, h_ref, wcat_ref, b_ref, ws_ref, wn_ref, bo_ref, out_ref):
        hqb = jnp.zeros((bn, D_FEAT), jnp.bfloat16)
        cp = jnp.zeros((bn, D_FEAT), jnp.float32)
        for t in range(K_NEI):
            x_cat = jnp.concatenate(
                [m_ref[t].astype(jnp.bfloat16), hqb], axis=1)
            g = jnp.dot(x_cat, wcat_ref[...],
                        preferred_element_type=jnp.float32) + b_ref[...]
            ti1 = jnp.tanh(g[:, :D_FEAT]) + 1.0
            tf1 = jnp.tanh(g[:, D_FEAT:2 * D_FEAT]) + 1.0
            gg = jnp.tanh(g[:, 2 * D_FEAT:3 * D_FEAT])
            to1 = jnp.tanh(g[:, 3 * D_FEAT:]) + 1.0
            if t == 0:
                cp = 0.5 * (gg * ti1)
            else:
                cp = 0.5 * (cp * tf1 + gg * ti1)
            hq = jnp.tanh(cp) * to1
            hqb = hq.astype(jnp.bfloat16)

        out = (
            jnp.dot(h_ref[...].astype(jnp.bfloat16), ws_ref[...],
                    preferred_element_type=jnp.float32)
            + jnp.dot(hqb, wn_ref[...], preferred_element_type=jnp.float32)
            + bo_ref[...]
        )
        if relu:
            out = jnp.maximum(out, 0.0)
        out_ref[...] = out

    return body


def _tc_layer(m_knd, h, wcat, b2, wsT, wnT, bo2, relu, bn=400):
    grid = (_NODES_C // bn,)
    full = lambda j: (0, 0)
    return pl.pallas_call(
        _make_lstm_body(bn, relu),
        grid=grid,
        in_specs=[
            pl.BlockSpec((K_NEI, bn, D_FEAT), lambda j: (0, j, 0)),
            pl.BlockSpec((bn, D_FEAT), lambda j: (j, 0)),
            pl.BlockSpec((2 * D_FEAT, 4 * D_FEAT), full),
            pl.BlockSpec((1, 4 * D_FEAT), full),
            pl.BlockSpec((D_FEAT, D_FEAT), full),
            pl.BlockSpec((D_FEAT, D_FEAT), full),
            pl.BlockSpec((1, D_FEAT), full),
        ],
        out_specs=pl.BlockSpec((bn, D_FEAT), lambda j: (j, 0)),
        out_shape=jax.ShapeDtypeStruct((_NODES_C, D_FEAT), jnp.float32),
        compiler_params=pltpu.CompilerParams(
            dimension_semantics=("arbitrary",)),
    )(m_knd, h, wcat, b2, wsT, wnT, bo2)


def kernel(x, edge_index, W_ih, W_hh, b_ih, b_hh, W_self, b_self, W_neigh, b_neigh):
    src = edge_index[0]
    idx4 = jnp.transpose(
        src.reshape(_NCH, _NODES_C, K_NEI), (0, 2, 1)
    ).reshape(_NCH, _NW, _N_CHUNKS, _CHUNK)

    cs = jnp.concatenate([
        jnp.full((D_FEAT,), 0.5, jnp.float32),
        jnp.full((D_FEAT,), 0.5, jnp.float32),
        jnp.ones((D_FEAT,), jnp.float32),
        jnp.full((D_FEAT,), 0.5, jnp.float32),
    ])
    wihT = jnp.transpose(W_ih, (0, 2, 1)) * cs
    whhT = jnp.transpose(W_hh, (0, 2, 1)) * (0.5 * cs)
    wcat = jnp.concatenate([wihT, whhT], axis=1).astype(jnp.bfloat16)
    b2 = ((b_ih + b_hh) * cs).reshape(L_LAYERS, 1, 4 * D_FEAT)
    wsT = jnp.transpose(W_self, (0, 2, 1)).astype(jnp.bfloat16)
    wnT = (0.5 * jnp.transpose(W_neigh, (0, 2, 1))).astype(jnp.bfloat16)
    bo2 = (b_self + b_neigh).reshape(L_LAYERS, 1, D_FEAT)

    h = x
    for l in range(L_LAYERS):
        relu = l < L_LAYERS - 1
        ms = [_sc_gather(h, idx4[c]) for c in range(_NCH)]
        outs = []
        for c in range(_NCH):
            m_knd = ms[c].reshape(K_NEI, _NODES_C, D_FEAT)
            h_c = lax.slice(h, (c * _NODES_C, 0), ((c + 1) * _NODES_C, D_FEAT))
            outs.append(_tc_layer(m_knd, h_c, wcat[l], b2[l],
                                  wsT[l], wnT[l], bo2[l], relu=relu))
        h = jnp.concatenate(outs, axis=0)
    return h


# speedup vs baseline: 7.4874x; 1.0579x over previous
import functools

import jax
import jax.numpy as jnp
from jax import lax
from jax.experimental import pallas as pl
from jax.experimental.pallas import tpu as pltpu
from jax.experimental.pallas import tpu_sc as plsc

N_NODES = 10000
K_NEI = 16
D_FEAT = 128
L_LAYERS = 4

_NC = 2
_NS = 16
_NW = _NC * _NS

_NCH = 5
_NODES_C = N_NODES // _NCH
_R_EDGES = _NODES_C * K_NEI
_PER_W = _R_EDGES // _NW
_CHUNK = 40
_N_CHUNKS = _PER_W // _CHUNK
_GROUP = 5
_N_GROUPS = _N_CHUNKS // _GROUP
_TOT_CHUNKS = _R_EDGES // _CHUNK


def _sc_gather(table, idx3):
    mesh = plsc.VectorSubcoreMesh(core_axis_name="c", subcore_axis_name="s")

    nslot = min(3, _N_GROUPS)

    @functools.partial(
        pl.kernel,
        mesh=mesh,
        out_type=jax.ShapeDtypeStruct((_TOT_CHUNKS, _CHUNK, D_FEAT), jnp.float32),
        scratch_types=[
            pltpu.VMEM((_N_CHUNKS, _CHUNK), jnp.int32),
            pltpu.VMEM((nslot, _GROUP, _CHUNK, D_FEAT), jnp.float32),
            [pltpu.SemaphoreType.DMA] * nslot,
        ],
    )
    def gather_kernel(table_hbm, idx_hbm, out_hbm, idx_v, rows_v, sems):
        wid = lax.axis_index("s") * _NC + lax.axis_index("c")
        cbase = wid * _N_CHUNKS
        pltpu.sync_copy(idx_hbm.at[wid], idx_v)

        def fire(g):
            s = g % nslot
            return [
                pltpu.async_copy(
                    table_hbm.at[idx_v.at[g * _GROUP + j]],
                    rows_v.at[s].at[j], sems[s])
                for j in range(_GROUP)
            ]

        handles = {g: fire(g) for g in range(nslot)}
        for g in range(_N_GROUPS):
            for h in handles.pop(g):
                h.wait()
            pltpu.sync_copy(rows_v.at[g % nslot],
                            out_hbm.at[pl.ds(cbase + g * _GROUP, _GROUP)])
            if g + nslot < _N_GROUPS:
                handles[g + nslot] = fire(g + nslot)

    return gather_kernel(table, idx3)


def _make_lstm_body(bn, relu):
    def body(m_ref, h_ref, wcat_ref, b_ref, ws_ref, wn_ref, bo_ref, out_ref):
        hqb = jnp.zeros((bn, D_FEAT), jnp.bfloat16)
        cp = jnp.zeros((bn, D_FEAT), jnp.float32)
        for t in range(K_NEI):
            x_cat = jnp.concatenate(
                [m_ref[t].astype(jnp.bfloat16), hqb], axis=1)
            g = jnp.dot(x_cat, wcat_ref[...],
                        preferred_element_type=jnp.float32) + b_ref[...]
            ti1 = jnp.tanh(g[:, :D_FEAT]) + 1.0
            tf1 = jnp.tanh(g[:, D_FEAT:2 * D_FEAT]) + 1.0
            gg = jnp.tanh(g[:, 2 * D_FEAT:3 * D_FEAT])
            to1 = jnp.tanh(g[:, 3 * D_FEAT:]) + 1.0
            if t == 0:
                cp = 0.5 * (gg * ti1)
            else:
                cp = 0.5 * (cp * tf1 + gg * ti1)
            hq = jnp.tanh(cp) * to1
            hqb = hq.astype(jnp.bfloat16)

        out = (
            jnp.dot(h_ref[...].astype(jnp.bfloat16), ws_ref[...],
                    preferred_element_type=jnp.float32)
            + jnp.dot(hqb, wn_ref[...], preferred_element_type=jnp.float32)
            + bo_ref[...]
        )
        if relu:
            out = jnp.maximum(out, 0.0)
        out_ref[...] = out

    return body


def _tc_layer(m_knd, h, wcat, b2, wsT, wnT, bo2, relu, bn=400):
    grid = (_NODES_C // bn,)
    full = lambda j: (0, 0)
    return pl.pallas_call(
        _make_lstm_body(bn, relu),
        grid=grid,
        in_specs=[
            pl.BlockSpec((K_NEI, bn, D_FEAT), lambda j: (0, j, 0)),
            pl.BlockSpec((bn, D_FEAT), lambda j: (j, 0)),
            pl.BlockSpec((2 * D_FEAT, 4 * D_FEAT), full),
            pl.BlockSpec((1, 4 * D_FEAT), full),
            pl.BlockSpec((D_FEAT, D_FEAT), full),
            pl.BlockSpec((D_FEAT, D_FEAT), full),
            pl.BlockSpec((1, D_FEAT), full),
        ],
        out_specs=pl.BlockSpec((bn, D_FEAT), lambda j: (j, 0)),
        out_shape=jax.ShapeDtypeStruct((_NODES_C, D_FEAT), jnp.float32),
        compiler_params=pltpu.CompilerParams(
            dimension_semantics=("arbitrary",)),
    )(m_knd, h, wcat, b2, wsT, wnT, bo2)


def kernel(x, edge_index, W_ih, W_hh, b_ih, b_hh, W_self, b_self, W_neigh, b_neigh):
    src = edge_index[0]
    idx4 = jnp.transpose(
        src.reshape(_NCH, _NODES_C, K_NEI), (0, 2, 1)
    ).reshape(_NCH, _NW, _N_CHUNKS, _CHUNK)

    cs = jnp.concatenate([
        jnp.full((D_FEAT,), 0.5, jnp.float32),
        jnp.full((D_FEAT,), 0.5, jnp.float32),
        jnp.ones((D_FEAT,), jnp.float32),
        jnp.full((D_FEAT,), 0.5, jnp.float32),
    ])
    wihT = jnp.transpose(W_ih, (0, 2, 1)) * cs
    whhT = jnp.transpose(W_hh, (0, 2, 1)) * (0.5 * cs)
    wcat = jnp.concatenate([wihT, whhT], axis=1).astype(jnp.bfloat16)
    b2 = ((b_ih + b_hh) * cs).reshape(L_LAYERS, 1, 4 * D_FEAT)
    wsT = jnp.transpose(W_self, (0, 2, 1)).astype(jnp.bfloat16)
    wnT = (0.5 * jnp.transpose(W_neigh, (0, 2, 1))).astype(jnp.bfloat16)
    bo2 = (b_self + b_neigh).reshape(L_LAYERS, 1, D_FEAT)

    h = x
    for l in range(L_LAYERS):
        relu = l < L_LAYERS - 1
        ms = [_sc_gather(h, idx4[c]) for c in range(_NCH)]
        outs = []
        for c in range(_NCH):
            m_knd = ms[c].reshape(K_NEI, _NODES_C, D_FEAT)
            h_c = lax.slice(h, (c * _NODES_C, 0), ((c + 1) * _NODES_C, D_FEAT))
            outs.append(_tc_layer(m_knd, h_c, wcat[l], b2[l],
                                  wsT[l], wnT[l], bo2[l], relu=relu))
        h = jnp.concatenate(outs, axis=0)
    return h


# speedup vs baseline: 7.9927x; 1.0675x over previous
import functools

import jax
import jax.numpy as jnp
from jax import lax
from jax.experimental import pallas as pl
from jax.experimental.pallas import tpu as pltpu
from jax.experimental.pallas import tpu_sc as plsc

N_NODES = 10000
K_NEI = 16
D_FEAT = 128
L_LAYERS = 4

_NC = 2
_NS = 16
_NW = _NC * _NS

_NCH = 5
_NODES_C = N_NODES // _NCH
_R_EDGES = _NODES_C * K_NEI
_PER_W = _R_EDGES // _NW
_CHUNK = 40
_N_CHUNKS = _PER_W // _CHUNK
_GROUP = 5
_N_GROUPS = _N_CHUNKS // _GROUP
_TOT_CHUNKS = _R_EDGES // _CHUNK


def _sc_gather(table, idx3):
    mesh = plsc.VectorSubcoreMesh(core_axis_name="c", subcore_axis_name="s")

    nslot = min(4, _N_GROUPS)

    @functools.partial(
        pl.kernel,
        mesh=mesh,
        out_type=jax.ShapeDtypeStruct((_TOT_CHUNKS, _CHUNK, D_FEAT), jnp.float32),
        scratch_types=[
            pltpu.VMEM((_N_CHUNKS, _CHUNK), jnp.int32),
            pltpu.VMEM((nslot, _GROUP, _CHUNK, D_FEAT), jnp.float32),
            [pltpu.SemaphoreType.DMA] * nslot,
        ],
    )
    def gather_kernel(table_hbm, idx_hbm, out_hbm, idx_v, rows_v, sems):
        wid = lax.axis_index("s") * _NC + lax.axis_index("c")
        cbase = wid * _N_CHUNKS
        pltpu.sync_copy(idx_hbm.at[wid], idx_v)

        def fire(g):
            s = g % nslot
            return [
                pltpu.async_copy(
                    table_hbm.at[idx_v.at[g * _GROUP + j]],
                    rows_v.at[s].at[j], sems[s])
                for j in range(_GROUP)
            ]

        handles = {g: fire(g) for g in range(nslot)}
        for g in range(_N_GROUPS):
            for h in handles.pop(g):
                h.wait()
            pltpu.sync_copy(rows_v.at[g % nslot],
                            out_hbm.at[pl.ds(cbase + g * _GROUP, _GROUP)])
            if g + nslot < _N_GROUPS:
                handles[g + nslot] = fire(g + nslot)

    return gather_kernel(table, idx3)


def _make_lstm_body(bn, relu):
    def body(m_ref, h_ref, wcat_ref, b_ref, ws_ref, wn_ref, bo_ref, out_ref):
        hqb = jnp.zeros((bn, D_FEAT), jnp.bfloat16)
        cp = jnp.zeros((bn, D_FEAT), jnp.float32)
        for t in range(K_NEI):
            x_cat = jnp.concatenate(
                [m_ref[t].astype(jnp.bfloat16), hqb], axis=1)
            g = jnp.dot(x_cat, wcat_ref[...],
                        preferred_element_type=jnp.float32) + b_ref[...]
            ti1 = jnp.tanh(g[:, :D_FEAT]) + 1.0
            tf1 = jnp.tanh(g[:, D_FEAT:2 * D_FEAT]) + 1.0
            gg = jnp.tanh(g[:, 2 * D_FEAT:3 * D_FEAT])
            to1 = jnp.tanh(g[:, 3 * D_FEAT:]) + 1.0
            if t == 0:
                cp = 0.5 * (gg * ti1)
            else:
                cp = 0.5 * (cp * tf1 + gg * ti1)
            hq = jnp.tanh(cp) * to1
            hqb = hq.astype(jnp.bfloat16)

        out = (
            jnp.dot(h_ref[...].astype(jnp.bfloat16), ws_ref[...],
                    preferred_element_type=jnp.float32)
            + jnp.dot(hqb, wn_ref[...], preferred_element_type=jnp.float32)
            + bo_ref[...]
        )
        if relu:
            out = jnp.maximum(out, 0.0)
        out_ref[...] = out

    return body


def _tc_layer(m_knd, h, wcat, b2, wsT, wnT, bo2, relu, c, bn=400):
    grid = (_NODES_C // bn,)
    full = lambda j: (0, 0)
    coff = c * (_NODES_C // bn)
    return pl.pallas_call(
        _make_lstm_body(bn, relu),
        grid=grid,
        in_specs=[
            pl.BlockSpec((K_NEI, bn, D_FEAT), lambda j: (0, j, 0)),
            pl.BlockSpec((bn, D_FEAT), lambda j: (coff + j, 0)),
            pl.BlockSpec((2 * D_FEAT, 4 * D_FEAT), full),
            pl.BlockSpec((1, 4 * D_FEAT), full),
            pl.BlockSpec((D_FEAT, D_FEAT), full),
            pl.BlockSpec((D_FEAT, D_FEAT), full),
            pl.BlockSpec((1, D_FEAT), full),
        ],
        out_specs=pl.BlockSpec((bn, D_FEAT), lambda j: (j, 0)),
        out_shape=jax.ShapeDtypeStruct((_NODES_C, D_FEAT), jnp.float32),
        compiler_params=pltpu.CompilerParams(
            dimension_semantics=("arbitrary",)),
    )(m_knd, h, wcat, b2, wsT, wnT, bo2)


def kernel(x, edge_index, W_ih, W_hh, b_ih, b_hh, W_self, b_self, W_neigh, b_neigh):
    src = edge_index[0]
    idx4 = jnp.transpose(
        src.reshape(_NCH, _NODES_C, K_NEI), (0, 2, 1)
    ).reshape(_NCH, _NW, _N_CHUNKS, _CHUNK)

    cs = jnp.concatenate([
        jnp.full((D_FEAT,), 0.5, jnp.float32),
        jnp.full((D_FEAT,), 0.5, jnp.float32),
        jnp.ones((D_FEAT,), jnp.float32),
        jnp.full((D_FEAT,), 0.5, jnp.float32),
    ])
    wihT = jnp.transpose(W_ih, (0, 2, 1)) * cs
    whhT = jnp.transpose(W_hh, (0, 2, 1)) * (0.5 * cs)
    wcat = jnp.concatenate([wihT, whhT], axis=1).astype(jnp.bfloat16)
    b2 = ((b_ih + b_hh) * cs).reshape(L_LAYERS, 1, 4 * D_FEAT)
    wsT = jnp.transpose(W_self, (0, 2, 1)).astype(jnp.bfloat16)
    wnT = (0.5 * jnp.transpose(W_neigh, (0, 2, 1))).astype(jnp.bfloat16)
    bo2 = (b_self + b_neigh).reshape(L_LAYERS, 1, D_FEAT)

    h = x
    for l in range(L_LAYERS):
        relu = l < L_LAYERS - 1
        ms = [_sc_gather(h, idx4[c]) for c in range(_NCH)]
        outs = []
        for c in range(_NCH):
            m_knd = ms[c].reshape(K_NEI, _NODES_C, D_FEAT)
            outs.append(_tc_layer(m_knd, h, wcat[l], b2[l],
                                  wsT[l], wnT[l], bo2[l], relu=relu, c=c))
        h = jnp.concatenate(outs, axis=0)
    return h


# speedup vs baseline: 8.0001x; 1.0009x over previous
import functools

import jax
import jax.numpy as jnp
from jax import lax
from jax.experimental import pallas as pl
from jax.experimental.pallas import tpu as pltpu
from jax.experimental.pallas import tpu_sc as plsc

N_NODES = 10000
K_NEI = 16
D_FEAT = 128
L_LAYERS = 4

_NC = 2
_NS = 16
_NW = _NC * _NS

_NCH = 5
_NODES_C = N_NODES // _NCH
_R_EDGES = _NODES_C * K_NEI
_PER_W = _R_EDGES // _NW
_CHUNK = 40
_N_CHUNKS = _PER_W // _CHUNK
_GROUP = 5
_N_GROUPS = _N_CHUNKS // _GROUP
_TOT_CHUNKS = _R_EDGES // _CHUNK


def _sc_gather(table, idx3):
    mesh = plsc.VectorSubcoreMesh(core_axis_name="c", subcore_axis_name="s")

    nslot = min(4, _N_GROUPS)

    @functools.partial(
        pl.kernel,
        mesh=mesh,
        out_type=jax.ShapeDtypeStruct((_TOT_CHUNKS, _CHUNK, D_FEAT), jnp.float32),
        scratch_types=[
            pltpu.VMEM((_N_CHUNKS, _CHUNK), jnp.int32),
            pltpu.VMEM((nslot, _GROUP, _CHUNK, D_FEAT), jnp.float32),
            [pltpu.SemaphoreType.DMA] * nslot,
        ],
    )
    def gather_kernel(table_hbm, idx_hbm, out_hbm, idx_v, rows_v, sems):
        wid = lax.axis_index("s") * _NC + lax.axis_index("c")
        cbase = wid * _N_CHUNKS
        pltpu.sync_copy(idx_hbm.at[wid], idx_v)

        def fire(g):
            s = g % nslot
            return [
                pltpu.async_copy(
                    table_hbm.at[idx_v.at[g * _GROUP + j]],
                    rows_v.at[s].at[j], sems[s])
                for j in range(_GROUP)
            ]

        handles = {g: fire(g) for g in range(nslot)}
        for g in range(_N_GROUPS):
            for h in handles.pop(g):
                h.wait()
            pltpu.sync_copy(rows_v.at[g % nslot],
                            out_hbm.at[pl.ds(cbase + g * _GROUP, _GROUP)])
            if g + nslot < _N_GROUPS:
                handles[g + nslot] = fire(g + nslot)

    return gather_kernel(table, idx3)


def _make_lstm_body(bn, relu):
    def body(m_ref, h_ref, wcat_ref, b_ref, ws_ref, wn_ref, bo_ref, out_ref):
        hqb = jnp.zeros((bn, D_FEAT), jnp.bfloat16)
        cp = jnp.zeros((bn, D_FEAT), jnp.float32)
        for t in range(K_NEI):
            x_cat = jnp.concatenate(
                [m_ref[t].astype(jnp.bfloat16), hqb], axis=1)
            g = jnp.dot(x_cat, wcat_ref[...],
                        preferred_element_type=jnp.float32) + b_ref[...]
            ti1 = jnp.tanh(g[:, :D_FEAT]) + 1.0
            tf1 = jnp.tanh(g[:, D_FEAT:2 * D_FEAT]) + 1.0
            gg = jnp.tanh(g[:, 2 * D_FEAT:3 * D_FEAT])
            to1 = jnp.tanh(g[:, 3 * D_FEAT:]) + 1.0
            if t == 0:
                cp = 0.5 * (gg * ti1)
            else:
                cp = 0.5 * (cp * tf1 + gg * ti1)
            hq = jnp.tanh(cp) * to1
            hqb = hq.astype(jnp.bfloat16)

        out = (
            jnp.dot(h_ref[...].astype(jnp.bfloat16), ws_ref[...],
                    preferred_element_type=jnp.float32)
            + jnp.dot(hqb, wn_ref[...], preferred_element_type=jnp.float32)
            + bo_ref[...]
        )
        if relu:
            out = jnp.maximum(out, 0.0)
        out_ref[...] = out

    return body


def _tc_layer(m_knd, h, wcat, b2, wsT, wnT, bo2, relu, c, bn=400):
    grid = (_NODES_C // bn,)
    full = lambda j: (0, 0)
    coff = c * (_NODES_C // bn)
    return pl.pallas_call(
        _make_lstm_body(bn, relu),
        grid=grid,
        in_specs=[
            pl.BlockSpec((K_NEI, bn, D_FEAT), lambda j: (0, j, 0)),
            pl.BlockSpec((bn, D_FEAT), lambda j: (coff + j, 0)),
            pl.BlockSpec((2 * D_FEAT, 4 * D_FEAT), full),
            pl.BlockSpec((1, 4 * D_FEAT), full),
            pl.BlockSpec((D_FEAT, D_FEAT), full),
            pl.BlockSpec((D_FEAT, D_FEAT), full),
            pl.BlockSpec((1, D_FEAT), full),
        ],
        out_specs=pl.BlockSpec((bn, D_FEAT), lambda j: (j, 0)),
        out_shape=jax.ShapeDtypeStruct((_NODES_C, D_FEAT), jnp.float32),
        compiler_params=pltpu.CompilerParams(
            dimension_semantics=("parallel",)),
    )(m_knd, h, wcat, b2, wsT, wnT, bo2)


def kernel(x, edge_index, W_ih, W_hh, b_ih, b_hh, W_self, b_self, W_neigh, b_neigh):
    src = edge_index[0]
    idx4 = jnp.transpose(
        src.reshape(_NCH, _NODES_C, K_NEI), (0, 2, 1)
    ).reshape(_NCH, _NW, _N_CHUNKS, _CHUNK)

    cs = jnp.concatenate([
        jnp.full((D_FEAT,), 0.5, jnp.float32),
        jnp.full((D_FEAT,), 0.5, jnp.float32),
        jnp.ones((D_FEAT,), jnp.float32),
        jnp.full((D_FEAT,), 0.5, jnp.float32),
    ])
    wihT = jnp.transpose(W_ih, (0, 2, 1)) * cs
    whhT = jnp.transpose(W_hh, (0, 2, 1)) * (0.5 * cs)
    wcat = jnp.concatenate([wihT, whhT], axis=1).astype(jnp.bfloat16)
    b2 = ((b_ih + b_hh) * cs).reshape(L_LAYERS, 1, 4 * D_FEAT)
    wsT = jnp.transpose(W_self, (0, 2, 1)).astype(jnp.bfloat16)
    wnT = (0.5 * jnp.transpose(W_neigh, (0, 2, 1))).astype(jnp.bfloat16)
    bo2 = (b_self + b_neigh).reshape(L_LAYERS, 1, D_FEAT)

    h = x
    for l in range(L_LAYERS):
        relu = l < L_LAYERS - 1
        ms = [_sc_gather(h, idx4[c]) for c in range(_NCH)]
        outs = []
        for c in range(_NCH):
            m_knd = ms[c].reshape(K_NEI, _NODES_C, D_FEAT)
            outs.append(_tc_layer(m_knd, h, wcat[l], b2[l],
                                  wsT[l], wnT[l], bo2[l], relu=relu, c=c))
        h = jnp.concatenate(outs, axis=0)
    return h
